# baseline (speedup 1.0000x reference)
import functools

import jax
import jax.numpy as jnp
from jax import lax
from jax.experimental import pallas as pl
from jax.experimental.pallas import tpu as pltpu
from jax.experimental.pallas import tpu_sc as plsc

DIM = 32
NUM_WORKERS = 32
CHUNK = 512
NBUF = 2
STRIPE = 1000
T_UNROLL = 8

_MESH = plsc.VectorSubcoreMesh(core_axis_name="c", subcore_axis_name="s")
_PARAMS = pltpu.CompilerParams(use_tc_tiling_on_sc=False)
_PARAMS_NOLAYOUT = pltpu.CompilerParams(use_tc_tiling_on_sc=False,
                                        needs_layout_passes=False)


@functools.partial(jax.jit, static_argnums=(2,))
def _transpose_tables(t0t, t1t, vocab):
    n_stripes = vocab // STRIPE
    assert vocab % STRIPE == 0 and STRIPE % T_UNROLL == 0

    @functools.partial(
        pl.kernel,
        mesh=_MESH,
        compiler_params=_PARAMS_NOLAYOUT,
        out_type=(
            jax.ShapeDtypeStruct((vocab, DIM), jnp.float32),
            jax.ShapeDtypeStruct((vocab, DIM), jnp.float32),
        ),
        scratch_types=[
            pltpu.VMEM((DIM, STRIPE), jnp.float32),
            pltpu.VMEM((STRIPE, DIM), jnp.float32),
            pltpu.SemaphoreType.DMA,
        ],
    )
    def k(t0_hbm, t1_hbm, s0_hbm, s1_hbm, vin, vout, wsem):
        wid = lax.axis_index("s") * 2 + lax.axis_index("c")
        rows_a = jax.lax.iota(jnp.int32, 16)
        rows_b = rows_a + 16

        def do_table(src_hbm, dst_hbm):
            def stripe_body(it, carry):
                sid = wid + it * NUM_WORKERS

                @pl.when(sid < n_stripes)
                def _():
                    v0 = sid * STRIPE
                    pltpu.sync_copy(src_hbm.at[:, pl.ds(v0, STRIPE)], vin)

                    def row_body(g, c):
                        for u in range(T_UNROLL):
                            v = g * T_UNROLL + u
                            cols = jnp.full((16,), v, jnp.int32)
                            lanes = jnp.full((16,), 1, jnp.int32)
                            xa = plsc.load_gather(vin, [rows_a, cols])
                            xb = plsc.load_gather(vin, [rows_b, cols])
                            plsc.store_scatter(
                                vout, [cols, rows_a], xa)
                            plsc.store_scatter(
                                vout, [cols, rows_b], xb)
                            del lanes
                        return c

                    lax.fori_loop(0, STRIPE // T_UNROLL, row_body, 0,
                                  unroll=False)
                    pltpu.async_copy(vout, dst_hbm.at[pl.ds(v0, STRIPE)],
                                     wsem).wait()
                return carry

            lax.fori_loop(0, (n_stripes + NUM_WORKERS - 1) // NUM_WORKERS,
                          stripe_body, 0, unroll=False)

        do_table(t0_hbm, s0_hbm)
        do_table(t1_hbm, s1_hbm)

    return k(t0t, t1t)


@functools.partial(jax.jit, static_argnums=(3, 4))
def _combine(idx_flat, table0, table1, total, per_worker):
    n_chunks = per_worker // CHUNK
    assert per_worker % CHUNK == 0 and n_chunks % NBUF == 0

    row_bufs = [
        [pltpu.VMEM((CHUNK, DIM), jnp.float32) for _ in range(2)]
        for _ in range(NBUF)
    ]
    gather_sems = [pltpu.SemaphoreType.DMA for _ in range(NBUF)]
    write_sems = [pltpu.SemaphoreType.DMA for _ in range(NBUF)]

    @functools.partial(
        pl.kernel,
        mesh=_MESH,
        compiler_params=_PARAMS,
        out_type=jax.ShapeDtypeStruct((total, 2 * DIM), jnp.float32),
        scratch_types=[pltpu.VMEM((per_worker,), jnp.int32), row_bufs,
                       gather_sems, write_sems],
    )
    def k(idx_hbm, t0_hbm, t1_hbm, out_hbm, idx_v, rbufs, gsems, wsems):
        wid = lax.axis_index("s") * 2 + lax.axis_index("c")
        base_w = wid * per_worker
        pltpu.sync_copy(idx_hbm.at[pl.ds(base_w, per_worker)], idx_v)

        def fire_gathers(i, b):
            sl = idx_v.at[pl.ds(i * CHUNK, CHUNK)]
            pltpu.async_copy(t0_hbm.at[sl], rbufs[b][0], gsems[b])
            pltpu.async_copy(t1_hbm.at[sl], rbufs[b][1], gsems[b])

        def wait_gathers(i, b):
            pltpu.make_async_copy(t0_hbm.at[idx_v.at[pl.ds(0, CHUNK)]],
                                  rbufs[b][0], gsems[b]).wait()
            pltpu.make_async_copy(t1_hbm.at[idx_v.at[pl.ds(0, CHUNK)]],
                                  rbufs[b][1], gsems[b]).wait()

        def fire_writes(i, b):
            base = base_w + i * CHUNK
            pltpu.async_copy(rbufs[b][0],
                             out_hbm.at[pl.ds(base, CHUNK), pl.ds(0, DIM)],
                             wsems[b])
            pltpu.async_copy(rbufs[b][1],
                             out_hbm.at[pl.ds(base, CHUNK), pl.ds(DIM, DIM)],
                             wsems[b])

        def wait_writes(b):
            pltpu.make_async_copy(rbufs[b][0],
                                  out_hbm.at[pl.ds(0, CHUNK), pl.ds(0, DIM)],
                                  wsems[b]).wait()
            pltpu.make_async_copy(rbufs[b][1],
                                  out_hbm.at[pl.ds(0, CHUNK), pl.ds(DIM, DIM)],
                                  wsems[b]).wait()

        for b in range(NBUF):
            fire_gathers(b, b)

        def body(g, carry):
            for b in range(NBUF):
                i = g + b
                wait_gathers(i, b)
                fire_writes(i, b)
                wait_writes(b)
                fire_gathers(i + NBUF, b)
            return carry

        lax.fori_loop(0, (n_chunks - NBUF) // NBUF,
                      lambda t, c: body(t * NBUF, c), 0, unroll=False)

        g0 = n_chunks - NBUF
        for b in range(NBUF):
            i = g0 + b
            wait_gathers(i, b)
            fire_writes(i, b)
            wait_writes(b)

    return k(idx_flat, table0, table1)


def kernel(input, table0, table1):
    B, L = input.shape
    total = B * L
    vocab = table0.shape[0]
    idx_flat = input.T.reshape(total).astype(jnp.int32)
    s0, s1 = _transpose_tables(table0.T, table1.T, vocab)
    per_worker = total // NUM_WORKERS
    out = _combine(idx_flat, s0, s1, total, per_worker)
    return out.reshape(L, B, 2 * DIM).transpose(1, 0, 2)


# speedup vs baseline: 3.7518x; 3.7518x over previous
import functools

import jax
import jax.numpy as jnp
from jax import lax
from jax.experimental import pallas as pl
from jax.experimental.pallas import tpu as pltpu
from jax.experimental.pallas import tpu_sc as plsc

DIM = 32
NUM_WORKERS = 32
CHUNK = 512
NBUF = 2
STRIPE = 896
TAIL = 64

_MESH = plsc.VectorSubcoreMesh(core_axis_name="c", subcore_axis_name="s")
_PARAMS = pltpu.CompilerParams(use_tc_tiling_on_sc=False)
_PARAMS_NOLAYOUT = pltpu.CompilerParams(use_tc_tiling_on_sc=False,
                                        needs_layout_passes=False)


def _transpose_body(vin_ref, vout_ref, n_groups):
    rows_a = jax.lax.iota(jnp.int32, 16)
    rows_b = rows_a + 16
    st_cols = [jax.lax.iota(jnp.int32, 16) + u * 32 for u in range(4)]

    def group(g, carry):
        cols_v, rows_st = carry
        for u in range(4):
            xa = plsc.load_gather(vin_ref, [rows_a, cols_v])
            xb = plsc.load_gather(vin_ref, [rows_b, cols_v])
            plsc.store_scatter(vout_ref, [rows_st, st_cols[u]], xa)
            plsc.store_scatter(vout_ref,
                               [rows_st, st_cols[u] + 16], xb)
            cols_v = cols_v + 1
        return cols_v, rows_st + 1

    lax.fori_loop(0, n_groups, group,
                  (jnp.zeros((16,), jnp.int32), jnp.zeros((16,), jnp.int32)),
                  unroll=False)


def _transpose_tables(t0t, t1t, t0tail, t1tail, vocab):
    n_stripes = (vocab - TAIL) // STRIPE
    assert (vocab - TAIL) % STRIPE == 0 and STRIPE % 128 == 0
    vrows = STRIPE * DIM // 128
    trows = TAIL * DIM // 128
    n_iter = (n_stripes + NUM_WORKERS - 1) // NUM_WORKERS

    @functools.partial(
        pl.kernel,
        mesh=_MESH,
        compiler_params=pltpu.CompilerParams(needs_layout_passes=False),
        out_type=(
            jax.ShapeDtypeStruct((vocab * DIM // 128, 128), jnp.float32),
            jax.ShapeDtypeStruct((vocab * DIM // 128, 128), jnp.float32),
        ),
        scratch_types=[
            pltpu.VMEM((DIM, STRIPE), jnp.float32),
            pltpu.VMEM((DIM, STRIPE), jnp.float32),
            pltpu.VMEM((vrows, 128), jnp.float32),
            pltpu.VMEM((vrows, 128), jnp.float32),
            pltpu.VMEM((DIM, TAIL), jnp.float32),
            pltpu.VMEM((trows, 128), jnp.float32),
            pltpu.SemaphoreType.DMA,
            pltpu.SemaphoreType.DMA,
            pltpu.SemaphoreType.DMA,
            pltpu.SemaphoreType.DMA,
        ],
    )
    def k(t0_hbm, t1_hbm, tl0_hbm, tl1_hbm, s0_hbm, s1_hbm,
          vin0, vin1, vout0, vout1, vtin, vtout, rs0, rs1, ws0, ws1):
        wid = lax.axis_index("s") * 2 + lax.axis_index("c")
        vins, vouts, rsems, wsems = (vin0, vin1), (vout0, vout1), \
            (rs0, rs1), (ws0, ws1)

        def do_table(src_hbm, dst_hbm):
            def fire_read(it, b):
                sid = wid + it * NUM_WORKERS

                @pl.when(sid < n_stripes)
                def _():
                    pltpu.async_copy(
                        src_hbm.at[:, pl.ds(sid * STRIPE, STRIPE)],
                        vins[b], rsems[b])

            def wait_read(b):
                pltpu.make_async_copy(
                    src_hbm.at[:, pl.ds(0, STRIPE)], vins[b],
                    rsems[b]).wait()

            def wait_write(b):
                pltpu.make_async_copy(
                    vouts[b], dst_hbm.at[pl.ds(0, vrows)], wsems[b]).wait()

            fire_read(0, 0)
            fire_read(1, 1)

            def body(it, carry):
                for b in range(2):
                    j = it * 2 + b
                    sid = wid + j * NUM_WORKERS

                    @pl.when(sid < n_stripes)
                    def _():
                        wait_read(b)

                        @pl.when(j >= 2)
                        def _():
                            wait_write(b)

                        _transpose_body(vins[b], vouts[b], STRIPE // 4)
                        pltpu.async_copy(
                            vouts[b], dst_hbm.at[pl.ds(sid * vrows, vrows)],
                            wsems[b])
                        fire_read(j + 2, b)
                return carry

            lax.fori_loop(0, (n_iter + 1) // 2, body, 0, unroll=False)
            for j in range(max(0, n_iter - 3), n_iter):
                fired = wid + j * NUM_WORKERS < n_stripes
                fired_n2 = wid + (j + 2) * NUM_WORKERS < n_stripes

                @pl.when(jnp.logical_and(fired, jnp.logical_not(fired_n2)))
                def _():
                    wait_write(j % 2)

        do_table(t0_hbm, s0_hbm)
        do_table(t1_hbm, s1_hbm)

        @pl.when(wid == 0)
        def _():
            for tl_hbm, dst_hbm in ((tl0_hbm, s0_hbm), (tl1_hbm, s1_hbm)):
                pltpu.sync_copy(tl_hbm, vtin)
                _transpose_body(vtin, vtout, TAIL // 4)
                pltpu.sync_copy(
                    vtout,
                    dst_hbm.at[pl.ds((vocab - TAIL) * DIM // 128, trows)])

    return k(t0t, t1t, t0tail, t1tail)


@functools.partial(jax.jit, static_argnums=(3, 4))
def _combine(idx_flat, table0, table1, total, per_worker):
    n_chunks = per_worker // CHUNK
    assert per_worker % CHUNK == 0 and n_chunks % NBUF == 0

    row_bufs = [
        [pltpu.VMEM((CHUNK, DIM), jnp.float32) for _ in range(2)]
        for _ in range(NBUF)
    ]
    gather_sems = [pltpu.SemaphoreType.DMA for _ in range(NBUF)]
    write_sems = [pltpu.SemaphoreType.DMA for _ in range(NBUF)]

    @functools.partial(
        pl.kernel,
        mesh=_MESH,
        compiler_params=_PARAMS,
        out_type=jax.ShapeDtypeStruct((total, 2 * DIM), jnp.float32),
        scratch_types=[pltpu.VMEM((per_worker,), jnp.int32), row_bufs,
                       gather_sems, write_sems],
    )
    def k(idx_hbm, t0_hbm, t1_hbm, out_hbm, idx_v, rbufs, gsems, wsems):
        wid = lax.axis_index("s") * 2 + lax.axis_index("c")
        base_w = wid * per_worker
        pltpu.sync_copy(idx_hbm.at[pl.ds(base_w, per_worker)], idx_v)

        def fire_gathers(i, b):
            sl = idx_v.at[pl.ds(i * CHUNK, CHUNK)]
            pltpu.async_copy(t0_hbm.at[sl], rbufs[b][0], gsems[b])
            pltpu.async_copy(t1_hbm.at[sl], rbufs[b][1], gsems[b])

        def wait_gathers(i, b):
            pltpu.make_async_copy(t0_hbm.at[idx_v.at[pl.ds(0, CHUNK)]],
                                  rbufs[b][0], gsems[b]).wait()
            pltpu.make_async_copy(t1_hbm.at[idx_v.at[pl.ds(0, CHUNK)]],
                                  rbufs[b][1], gsems[b]).wait()

        def fire_writes(i, b):
            base = base_w + i * CHUNK
            pltpu.async_copy(rbufs[b][0],
                             out_hbm.at[pl.ds(base, CHUNK), pl.ds(0, DIM)],
                             wsems[b])
            pltpu.async_copy(rbufs[b][1],
                             out_hbm.at[pl.ds(base, CHUNK), pl.ds(DIM, DIM)],
                             wsems[b])

        def wait_writes(b):
            pltpu.make_async_copy(rbufs[b][0],
                                  out_hbm.at[pl.ds(0, CHUNK), pl.ds(0, DIM)],
                                  wsems[b]).wait()
            pltpu.make_async_copy(rbufs[b][1],
                                  out_hbm.at[pl.ds(0, CHUNK), pl.ds(DIM, DIM)],
                                  wsems[b]).wait()

        for b in range(NBUF):
            fire_gathers(b, b)

        def body(g, carry):
            for b in range(NBUF):
                i = g + b
                wait_gathers(i, b)
                fire_writes(i, b)
                wait_writes(b)
                fire_gathers(i + NBUF, b)
            return carry

        lax.fori_loop(0, (n_chunks - NBUF) // NBUF,
                      lambda t, c: body(t * NBUF, c), 0, unroll=False)

        g0 = n_chunks - NBUF
        for b in range(NBUF):
            i = g0 + b
            wait_gathers(i, b)
            fire_writes(i, b)
            wait_writes(b)

    return k(idx_flat, table0, table1)


def kernel(input, table0, table1):
    B, L = input.shape
    total = B * L
    vocab = table0.shape[0]
    idx_flat = input.T.reshape(total).astype(jnp.int32)
    s0f, s1f = _transpose_tables(table0.T, table1.T,
                                 table0[vocab - TAIL:].T,
                                 table1[vocab - TAIL:].T, vocab)
    s0 = s0f.reshape(vocab, DIM)
    s1 = s1f.reshape(vocab, DIM)
    per_worker = total // NUM_WORKERS
    out = _combine(idx_flat, s0, s1, total, per_worker)
    return out.reshape(L, B, 2 * DIM).transpose(1, 0, 2)


# speedup vs baseline: 3.7519x; 1.0000x over previous
import functools

import jax
import jax.numpy as jnp
from jax import lax
from jax.experimental import pallas as pl
from jax.experimental.pallas import tpu as pltpu
from jax.experimental.pallas import tpu_sc as plsc

DIM = 32
NUM_WORKERS = 32
CHUNK = 512
NBUF = 2
STRIPE = 896
TAIL = 64

_MESH = plsc.VectorSubcoreMesh(core_axis_name="c", subcore_axis_name="s")
_PARAMS = pltpu.CompilerParams(use_tc_tiling_on_sc=False)
_PARAMS_NOLAYOUT = pltpu.CompilerParams(use_tc_tiling_on_sc=False,
                                        needs_layout_passes=False)


def _transpose_body(vin_ref, vout_ref, n_groups):
    rows_a = jax.lax.iota(jnp.int32, 16)
    rows_b = rows_a + 16
    st_cols = [jax.lax.iota(jnp.int32, 16) + u * 32 for u in range(4)]

    def group(g, carry):
        cols_v, rows_st = carry
        for u in range(4):
            xa = plsc.load_gather(vin_ref, [rows_a, cols_v])
            xb = plsc.load_gather(vin_ref, [rows_b, cols_v])
            plsc.store_scatter(vout_ref, [rows_st, st_cols[u]], xa)
            plsc.store_scatter(vout_ref,
                               [rows_st, st_cols[u] + 16], xb)
            cols_v = cols_v + 1
        return cols_v, rows_st + 1

    lax.fori_loop(0, n_groups, group,
                  (jnp.zeros((16,), jnp.int32), jnp.zeros((16,), jnp.int32)),
                  unroll=False)


def _transpose_tables(t0t, t1t, t0tail, t1tail, vocab):
    n_stripes = (vocab - TAIL) // STRIPE
    assert (vocab - TAIL) % STRIPE == 0 and STRIPE % 128 == 0
    vrows = STRIPE * DIM // 128
    trows = TAIL * DIM // 128
    n_iter = (n_stripes + NUM_WORKERS - 1) // NUM_WORKERS

    @functools.partial(
        pl.kernel,
        mesh=_MESH,
        compiler_params=pltpu.CompilerParams(needs_layout_passes=False),
        out_type=(
            jax.ShapeDtypeStruct((vocab * DIM // 128, 128), jnp.float32),
            jax.ShapeDtypeStruct((vocab * DIM // 128, 128), jnp.float32),
        ),
        scratch_types=[
            pltpu.VMEM((DIM, STRIPE + 1), jnp.float32),
            pltpu.VMEM((DIM, STRIPE + 1), jnp.float32),
            pltpu.VMEM((vrows, 128), jnp.float32),
            pltpu.VMEM((vrows, 128), jnp.float32),
            pltpu.VMEM((DIM, TAIL), jnp.float32),
            pltpu.VMEM((trows, 128), jnp.float32),
            pltpu.SemaphoreType.DMA,
            pltpu.SemaphoreType.DMA,
            pltpu.SemaphoreType.DMA,
            pltpu.SemaphoreType.DMA,
        ],
    )
    def k(t0_hbm, t1_hbm, tl0_hbm, tl1_hbm, s0_hbm, s1_hbm,
          vin0, vin1, vout0, vout1, vtin, vtout, rs0, rs1, ws0, ws1):
        wid = lax.axis_index("s") * 2 + lax.axis_index("c")
        vins, vouts, rsems, wsems = (vin0, vin1), (vout0, vout1), \
            (rs0, rs1), (ws0, ws1)

        def do_table(src_hbm, dst_hbm):
            def fire_read(it, b):
                sid = wid + it * NUM_WORKERS

                @pl.when(sid < n_stripes)
                def _():
                    pltpu.async_copy(
                        src_hbm.at[:, pl.ds(sid * STRIPE, STRIPE)],
                        vins[b].at[:, pl.ds(0, STRIPE)], rsems[b])

            def wait_read(b):
                pltpu.make_async_copy(
                    src_hbm.at[:, pl.ds(0, STRIPE)],
                    vins[b].at[:, pl.ds(0, STRIPE)], rsems[b]).wait()

            def wait_write(b):
                pltpu.make_async_copy(
                    vouts[b], dst_hbm.at[pl.ds(0, vrows)], wsems[b]).wait()

            fire_read(0, 0)
            fire_read(1, 1)

            def body(it, carry):
                for b in range(2):
                    j = it * 2 + b
                    sid = wid + j * NUM_WORKERS

                    @pl.when(sid < n_stripes)
                    def _():
                        wait_read(b)

                        @pl.when(j >= 2)
                        def _():
                            wait_write(b)

                        _transpose_body(vins[b], vouts[b], STRIPE // 4)
                        pltpu.async_copy(
                            vouts[b], dst_hbm.at[pl.ds(sid * vrows, vrows)],
                            wsems[b])
                        fire_read(j + 2, b)
                return carry

            lax.fori_loop(0, (n_iter + 1) // 2, body, 0, unroll=False)
            for j in range(max(0, n_iter - 3), n_iter):
                fired = wid + j * NUM_WORKERS < n_stripes
                fired_n2 = wid + (j + 2) * NUM_WORKERS < n_stripes

                @pl.when(jnp.logical_and(fired, jnp.logical_not(fired_n2)))
                def _():
                    wait_write(j % 2)

        do_table(t0_hbm, s0_hbm)
        do_table(t1_hbm, s1_hbm)

        @pl.when(wid == 0)
        def _():
            for tl_hbm, dst_hbm in ((tl0_hbm, s0_hbm), (tl1_hbm, s1_hbm)):
                pltpu.sync_copy(tl_hbm, vtin)
                _transpose_body(vtin, vtout, TAIL // 4)
                pltpu.sync_copy(
                    vtout,
                    dst_hbm.at[pl.ds((vocab - TAIL) * DIM // 128, trows)])

    return k(t0t, t1t, t0tail, t1tail)


@functools.partial(jax.jit, static_argnums=(3, 4))
def _combine(idx_flat, table0, table1, total, per_worker):
    n_chunks = per_worker // CHUNK
    assert per_worker % CHUNK == 0 and n_chunks % NBUF == 0

    row_bufs = [
        [pltpu.VMEM((CHUNK, DIM), jnp.float32) for _ in range(2)]
        for _ in range(NBUF)
    ]
    gather_sems = [pltpu.SemaphoreType.DMA for _ in range(NBUF)]
    write_sems = [pltpu.SemaphoreType.DMA for _ in range(NBUF)]

    @functools.partial(
        pl.kernel,
        mesh=_MESH,
        compiler_params=_PARAMS,
        out_type=jax.ShapeDtypeStruct((total, 2 * DIM), jnp.float32),
        scratch_types=[pltpu.VMEM((per_worker,), jnp.int32), row_bufs,
                       gather_sems, write_sems],
    )
    def k(idx_hbm, t0_hbm, t1_hbm, out_hbm, idx_v, rbufs, gsems, wsems):
        wid = lax.axis_index("s") * 2 + lax.axis_index("c")
        base_w = wid * per_worker
        pltpu.sync_copy(idx_hbm.at[pl.ds(base_w, per_worker)], idx_v)

        def fire_gathers(i, b):
            sl = idx_v.at[pl.ds(i * CHUNK, CHUNK)]
            pltpu.async_copy(t0_hbm.at[sl], rbufs[b][0], gsems[b])
            pltpu.async_copy(t1_hbm.at[sl], rbufs[b][1], gsems[b])

        def wait_gathers(i, b):
            pltpu.make_async_copy(t0_hbm.at[idx_v.at[pl.ds(0, CHUNK)]],
                                  rbufs[b][0], gsems[b]).wait()
            pltpu.make_async_copy(t1_hbm.at[idx_v.at[pl.ds(0, CHUNK)]],
                                  rbufs[b][1], gsems[b]).wait()

        def fire_writes(i, b):
            base = base_w + i * CHUNK
            pltpu.async_copy(rbufs[b][0],
                             out_hbm.at[pl.ds(base, CHUNK), pl.ds(0, DIM)],
                             wsems[b])
            pltpu.async_copy(rbufs[b][1],
                             out_hbm.at[pl.ds(base, CHUNK), pl.ds(DIM, DIM)],
                             wsems[b])

        def wait_writes(b):
            pltpu.make_async_copy(rbufs[b][0],
                                  out_hbm.at[pl.ds(0, CHUNK), pl.ds(0, DIM)],
                                  wsems[b]).wait()
            pltpu.make_async_copy(rbufs[b][1],
                                  out_hbm.at[pl.ds(0, CHUNK), pl.ds(DIM, DIM)],
                                  wsems[b]).wait()

        for b in range(NBUF):
            fire_gathers(b, b)

        def body(g, carry):
            for b in range(NBUF):
                i = g + b
                wait_gathers(i, b)
                fire_writes(i, b)
                wait_writes(b)
                fire_gathers(i + NBUF, b)
            return carry

        lax.fori_loop(0, (n_chunks - NBUF) // NBUF,
                      lambda t, c: body(t * NBUF, c), 0, unroll=False)

        g0 = n_chunks - NBUF
        for b in range(NBUF):
            i = g0 + b
            wait_gathers(i, b)
            fire_writes(i, b)
            wait_writes(b)

    return k(idx_flat, table0, table1)


def kernel(input, table0, table1):
    B, L = input.shape
    total = B * L
    vocab = table0.shape[0]
    idx_flat = input.T.reshape(total).astype(jnp.int32)
    s0f, s1f = _transpose_tables(table0.T, table1.T,
                                 table0[vocab - TAIL:].T,
                                 table1[vocab - TAIL:].T, vocab)
    s0 = s0f.reshape(vocab, DIM)
    s1 = s1f.reshape(vocab, DIM)
    per_worker = total // NUM_WORKERS
    out = _combine(idx_flat, s0, s1, total, per_worker)
    return out.reshape(L, B, 2 * DIM).transpose(1, 0, 2)


# speedup vs baseline: 4.0160x; 1.0704x over previous
import functools

import jax
import jax.numpy as jnp
from jax import lax
from jax.experimental import pallas as pl
from jax.experimental.pallas import tpu as pltpu
from jax.experimental.pallas import tpu_sc as plsc

DIM = 32
NUM_WORKERS = 32
CHUNK = 512
NBUF = 2
STRIPE = 896
TAIL = 64

_MESH = plsc.VectorSubcoreMesh(core_axis_name="c", subcore_axis_name="s")
_PARAMS = pltpu.CompilerParams(use_tc_tiling_on_sc=False)
_PARAMS_NOLAYOUT = pltpu.CompilerParams(use_tc_tiling_on_sc=False,
                                        needs_layout_passes=False)


def _transpose_body(vin_ref, vout_ref, n_groups):
    iota = jax.lax.iota(jnp.int32, 16)
    cols_f = [(iota % 4) * DIM + f for f in range(DIM)]

    def group(g, carry):
        q_v = carry
        base = g * 16
        for f in range(DIM):
            x = vin_ref[f, pl.ds(base, 16)]
            plsc.store_scatter(vout_ref, [q_v, cols_f[f]], x)
        return q_v + 4

    lax.fori_loop(0, n_groups, group, jax.lax.iota(jnp.int32, 16) // 4,
                  unroll=False)


def _transpose_tables(t0t, t1t, t0tail, t1tail, vocab):
    n_stripes = (vocab - TAIL) // STRIPE
    assert (vocab - TAIL) % STRIPE == 0 and STRIPE % 16 == 0
    vrows = STRIPE // 4
    trows = TAIL // 4
    n_iter = (n_stripes + NUM_WORKERS - 1) // NUM_WORKERS

    @functools.partial(
        pl.kernel,
        mesh=_MESH,
        compiler_params=pltpu.CompilerParams(needs_layout_passes=False,
                                             disable_bounds_checks=True),
        out_type=(
            jax.ShapeDtypeStruct((vocab // 4, 4 * DIM), jnp.float32),
            jax.ShapeDtypeStruct((vocab // 4, 4 * DIM), jnp.float32),
        ),
        scratch_types=[
            pltpu.VMEM((DIM, STRIPE + 1), jnp.float32),
            pltpu.VMEM((DIM, STRIPE + 1), jnp.float32),
            pltpu.VMEM((vrows, 4 * DIM), jnp.float32),
            pltpu.VMEM((vrows, 4 * DIM), jnp.float32),
            pltpu.VMEM((DIM, TAIL), jnp.float32),
            pltpu.VMEM((trows, 4 * DIM), jnp.float32),
            pltpu.SemaphoreType.DMA,
            pltpu.SemaphoreType.DMA,
            pltpu.SemaphoreType.DMA,
            pltpu.SemaphoreType.DMA,
        ],
    )
    def k(t0_hbm, t1_hbm, tl0_hbm, tl1_hbm, s0_hbm, s1_hbm,
          vin0, vin1, vout0, vout1, vtin, vtout, rs0, rs1, ws0, ws1):
        wid = lax.axis_index("s") * 2 + lax.axis_index("c")
        vins, vouts, rsems, wsems = (vin0, vin1), (vout0, vout1), \
            (rs0, rs1), (ws0, ws1)

        def do_table(src_hbm, dst_hbm):
            def fire_read(it, b):
                sid = wid + it * NUM_WORKERS

                @pl.when(sid < n_stripes)
                def _():
                    pltpu.async_copy(
                        src_hbm.at[:, pl.ds(sid * STRIPE, STRIPE)],
                        vins[b].at[:, pl.ds(0, STRIPE)], rsems[b])

            def wait_read(b):
                pltpu.make_async_copy(
                    src_hbm.at[:, pl.ds(0, STRIPE)],
                    vins[b].at[:, pl.ds(0, STRIPE)], rsems[b]).wait()

            def wait_write(b):
                pltpu.make_async_copy(
                    vouts[b], dst_hbm.at[pl.ds(0, vrows)], wsems[b]).wait()

            fire_read(0, 0)
            fire_read(1, 1)

            def body(it, carry):
                for b in range(2):
                    j = it * 2 + b
                    sid = wid + j * NUM_WORKERS

                    @pl.when(sid < n_stripes)
                    def _():
                        wait_read(b)

                        @pl.when(j >= 2)
                        def _():
                            wait_write(b)

                        _transpose_body(vins[b], vouts[b], STRIPE // 16)
                        pltpu.async_copy(
                            vouts[b],
                            dst_hbm.at[pl.ds(sid * vrows, vrows)],
                            wsems[b])
                        fire_read(j + 2, b)
                return carry

            lax.fori_loop(0, (n_iter + 1) // 2, body, 0, unroll=False)
            for j in range(max(0, n_iter - 3), n_iter):
                fired = wid + j * NUM_WORKERS < n_stripes
                fired_n2 = wid + (j + 2) * NUM_WORKERS < n_stripes

                @pl.when(jnp.logical_and(fired, jnp.logical_not(fired_n2)))
                def _():
                    wait_write(j % 2)

        do_table(t0_hbm, s0_hbm)
        do_table(t1_hbm, s1_hbm)

        @pl.when(wid == 0)
        def _():
            for tl_hbm, dst_hbm in ((tl0_hbm, s0_hbm), (tl1_hbm, s1_hbm)):
                pltpu.sync_copy(tl_hbm, vtin)
                _transpose_body(vtin, vtout, TAIL // 16)
                pltpu.sync_copy(
                    vtout,
                    dst_hbm.at[pl.ds((vocab - TAIL) // 4, trows)])

    return k(t0t, t1t, t0tail, t1tail)


@functools.partial(jax.jit, static_argnums=(3, 4))
def _combine(idx_flat, table0, table1, total, per_worker):
    n_chunks = per_worker // CHUNK
    assert per_worker % CHUNK == 0 and n_chunks % NBUF == 0

    row_bufs = [
        [pltpu.VMEM((CHUNK, DIM), jnp.float32) for _ in range(2)]
        for _ in range(NBUF)
    ]
    gather_sems = [pltpu.SemaphoreType.DMA for _ in range(NBUF)]
    write_sems = [pltpu.SemaphoreType.DMA for _ in range(NBUF)]

    @functools.partial(
        pl.kernel,
        mesh=_MESH,
        compiler_params=_PARAMS,
        out_type=jax.ShapeDtypeStruct((total, 2 * DIM), jnp.float32),
        scratch_types=[pltpu.VMEM((per_worker,), jnp.int32), row_bufs,
                       gather_sems, write_sems],
    )
    def k(idx_hbm, t0_hbm, t1_hbm, out_hbm, idx_v, rbufs, gsems, wsems):
        wid = lax.axis_index("s") * 2 + lax.axis_index("c")
        base_w = wid * per_worker
        pltpu.sync_copy(idx_hbm.at[pl.ds(base_w, per_worker)], idx_v)

        def fire_gathers(i, b):
            sl = idx_v.at[pl.ds(i * CHUNK, CHUNK)]
            pltpu.async_copy(t0_hbm.at[sl], rbufs[b][0], gsems[b])
            pltpu.async_copy(t1_hbm.at[sl], rbufs[b][1], gsems[b])

        def wait_gathers(i, b):
            pltpu.make_async_copy(t0_hbm.at[idx_v.at[pl.ds(0, CHUNK)]],
                                  rbufs[b][0], gsems[b]).wait()
            pltpu.make_async_copy(t1_hbm.at[idx_v.at[pl.ds(0, CHUNK)]],
                                  rbufs[b][1], gsems[b]).wait()

        def fire_writes(i, b):
            base = base_w + i * CHUNK
            pltpu.async_copy(rbufs[b][0],
                             out_hbm.at[pl.ds(base, CHUNK), pl.ds(0, DIM)],
                             wsems[b])
            pltpu.async_copy(rbufs[b][1],
                             out_hbm.at[pl.ds(base, CHUNK), pl.ds(DIM, DIM)],
                             wsems[b])

        def wait_writes(b):
            pltpu.make_async_copy(rbufs[b][0],
                                  out_hbm.at[pl.ds(0, CHUNK), pl.ds(0, DIM)],
                                  wsems[b]).wait()
            pltpu.make_async_copy(rbufs[b][1],
                                  out_hbm.at[pl.ds(0, CHUNK), pl.ds(DIM, DIM)],
                                  wsems[b]).wait()

        for b in range(NBUF):
            fire_gathers(b, b)

        def body(g, carry):
            for b in range(NBUF):
                i = g + b
                wait_gathers(i, b)
                fire_writes(i, b)
                wait_writes(b)
                fire_gathers(i + NBUF, b)
            return carry

        lax.fori_loop(0, (n_chunks - NBUF) // NBUF,
                      lambda t, c: body(t * NBUF, c), 0, unroll=False)

        g0 = n_chunks - NBUF
        for b in range(NBUF):
            i = g0 + b
            wait_gathers(i, b)
            fire_writes(i, b)
            wait_writes(b)

    return k(idx_flat, table0, table1)


def kernel(input, table0, table1):
    B, L = input.shape
    total = B * L
    vocab = table0.shape[0]
    idx_flat = input.T.reshape(total).astype(jnp.int32)
    s0f, s1f = _transpose_tables(table0.T, table1.T,
                                 table0[vocab - TAIL:].T,
                                 table1[vocab - TAIL:].T, vocab)
    s0 = s0f.reshape(vocab, DIM)
    s1 = s1f.reshape(vocab, DIM)
    per_worker = total // NUM_WORKERS
    out = _combine(idx_flat, s0, s1, total, per_worker)
    return out.reshape(L, B, 2 * DIM).transpose(1, 0, 2)


# speedup vs baseline: 5.3184x; 1.3243x over previous
import functools

import jax
import jax.numpy as jnp
from jax import lax
from jax.experimental import pallas as pl
from jax.experimental.pallas import tpu as pltpu
from jax.experimental.pallas import tpu_sc as plsc

DIM = 32
NUM_WORKERS = 32
CHUNK = 512
NBUF = 2

_MESH = plsc.VectorSubcoreMesh(core_axis_name="c", subcore_axis_name="s")
_PARAMS = pltpu.CompilerParams(use_tc_tiling_on_sc=False)


@functools.partial(jax.jit, static_argnums=(3, 4))
def _combine(idx_flat, table0, table1, total, per_worker):
    n_chunks = per_worker // CHUNK
    assert per_worker % CHUNK == 0 and n_chunks % NBUF == 0

    row_bufs = [
        [pltpu.VMEM((CHUNK, DIM), jnp.float32) for _ in range(2)]
        for _ in range(NBUF)
    ]
    gather_sems = [pltpu.SemaphoreType.DMA for _ in range(NBUF)]
    write_sems = [pltpu.SemaphoreType.DMA for _ in range(NBUF)]

    @functools.partial(
        pl.kernel,
        mesh=_MESH,
        compiler_params=_PARAMS,
        out_type=jax.ShapeDtypeStruct((total, 2 * DIM), jnp.float32),
        scratch_types=[pltpu.VMEM((per_worker,), jnp.int32), row_bufs,
                       gather_sems, write_sems],
    )
    def k(idx_hbm, t0_hbm, t1_hbm, out_hbm, idx_v, rbufs, gsems, wsems):
        wid = lax.axis_index("s") * 2 + lax.axis_index("c")
        base_w = wid * per_worker
        pltpu.sync_copy(idx_hbm.at[pl.ds(base_w, per_worker)], idx_v)

        def fire_gathers(i, b):
            sl = idx_v.at[pl.ds(i * CHUNK, CHUNK)]
            pltpu.async_copy(t0_hbm.at[sl], rbufs[b][0], gsems[b])
            pltpu.async_copy(t1_hbm.at[sl], rbufs[b][1], gsems[b])

        def wait_gathers(i, b):
            pltpu.make_async_copy(t0_hbm.at[idx_v.at[pl.ds(0, CHUNK)]],
                                  rbufs[b][0], gsems[b]).wait()
            pltpu.make_async_copy(t1_hbm.at[idx_v.at[pl.ds(0, CHUNK)]],
                                  rbufs[b][1], gsems[b]).wait()

        def fire_writes(i, b):
            base = base_w + i * CHUNK
            pltpu.async_copy(rbufs[b][0],
                             out_hbm.at[pl.ds(base, CHUNK), pl.ds(0, DIM)],
                             wsems[b])
            pltpu.async_copy(rbufs[b][1],
                             out_hbm.at[pl.ds(base, CHUNK), pl.ds(DIM, DIM)],
                             wsems[b])

        def wait_writes(b):
            pltpu.make_async_copy(rbufs[b][0],
                                  out_hbm.at[pl.ds(0, CHUNK), pl.ds(0, DIM)],
                                  wsems[b]).wait()
            pltpu.make_async_copy(rbufs[b][1],
                                  out_hbm.at[pl.ds(0, CHUNK), pl.ds(DIM, DIM)],
                                  wsems[b]).wait()

        for b in range(NBUF):
            fire_gathers(b, b)

        def body(g, carry):
            for b in range(NBUF):
                i = g + b
                wait_gathers(i, b)
                fire_writes(i, b)
                wait_writes(b)
                fire_gathers(i + NBUF, b)
            return carry

        lax.fori_loop(0, (n_chunks - NBUF) // NBUF,
                      lambda t, c: body(t * NBUF, c), 0, unroll=False)

        g0 = n_chunks - NBUF
        for b in range(NBUF):
            i = g0 + b
            wait_gathers(i, b)
            fire_writes(i, b)
            wait_writes(b)

    return k(idx_flat, table0, table1)


def kernel(input, table0, table1):
    B, L = input.shape
    total = B * L
    idx_flat = input.T.reshape(total).astype(jnp.int32)
    per_worker = total // NUM_WORKERS
    out = _combine(idx_flat, table0, table1, total, per_worker)
    return out.reshape(L, B, 2 * DIM).transpose(1, 0, 2)
